# Initial kernel scaffold; baseline (speedup 1.0000x reference)
import jax
import jax.numpy as jnp
from jax.experimental import pallas as pl


def kernel(x, protos, log_tau, log_gamma, log_blend):
    raise NotImplementedError("write your pallas kernel here")


# speedup vs baseline: 2.0070x; 2.0070x over previous
import math

import jax
import jax.numpy as jnp
from jax.experimental import pallas as pl
from jax.experimental.pallas import tpu as pltpu

_SQRT_2_OVER_PI = math.sqrt(2.0 / math.pi)


def _body(lt_ref, lg_ref, lb_ref, x_ref, p_ref, o_ref):
    tau = jnp.exp(lt_ref[0])
    gamma = jnp.exp(lg_ref[0])
    alpha = jax.nn.sigmoid(lb_ref[0])

    xb = x_ref[:]
    g = 0.5 * xb * (1.0 + jnp.tanh(_SQRT_2_OVER_PI * (xb + 0.044715 * xb * xb * xb)))

    p = p_ref[:]
    p_norm = jnp.sqrt(jnp.sum(p * p, axis=-1, keepdims=True))
    pn = p / jnp.maximum(p_norm, 1e-12)

    g_norm = jnp.sqrt(jnp.sum(g * g, axis=-1, keepdims=True))
    inv_gn = 1.0 / jnp.maximum(g_norm, 1e-12)

    sims = jnp.dot(g, pn.T, preferred_element_type=jnp.float32) * inv_gn

    z = sims * tau
    m = jnp.max(z, axis=-1, keepdims=True)
    lse = m[:, 0] + jnp.log(jnp.sum(jnp.exp(z - m), axis=-1))
    k = p.shape[0]
    soft = (lse - math.log(k)) / tau

    novelty = jnp.exp(-gamma * soft)
    gate = 1.0 - alpha + alpha * novelty
    o_ref[:] = g * gate[:, None]


def kernel(x, protos, log_tau, log_gamma, log_blend):
    B, T, D = x.shape
    K = protos.shape[0]
    rows = B * T
    x2 = x.reshape(rows, D)

    block_rows = 256
    grid = (rows // block_rows,)

    out = pl.pallas_call(
        _body,
        grid=grid,
        in_specs=[
            pl.BlockSpec(memory_space=pltpu.SMEM),
            pl.BlockSpec(memory_space=pltpu.SMEM),
            pl.BlockSpec(memory_space=pltpu.SMEM),
            pl.BlockSpec((block_rows, D), lambda i: (i, 0)),
            pl.BlockSpec((K, D), lambda i: (0, 0)),
        ],
        out_specs=pl.BlockSpec((block_rows, D), lambda i: (i, 0)),
        out_shape=jax.ShapeDtypeStruct((rows, D), x.dtype),
        compiler_params=pltpu.CompilerParams(
            dimension_semantics=("arbitrary",),
        ),
    )(
        log_tau.reshape(1),
        log_gamma.reshape(1),
        log_blend.reshape(1),
        x2,
        protos,
    )
    return out.reshape(B, T, D)


# speedup vs baseline: 2.2358x; 1.1140x over previous
import math

import jax
import jax.numpy as jnp
from jax.experimental import pallas as pl
from jax.experimental.pallas import tpu as pltpu

_SQRT_2_OVER_PI = math.sqrt(2.0 / math.pi)


_K1 = _SQRT_2_OVER_PI * 0.044715


def _body(lt_ref, lg_ref, lb_ref, x_ref, p_ref, o_ref):
    tau = jnp.exp(lt_ref[0])
    gamma = jnp.exp(lg_ref[0])
    alpha = jax.nn.sigmoid(lb_ref[0])

    xb = x_ref[:]
    x2 = xb * xb
    y = xb * (_K1 * x2 + _SQRT_2_OVER_PI)
    w = xb * (1.0 + jnp.tanh(y))
    w2 = w * w

    p = p_ref[:]
    p_norm = jnp.sqrt(jnp.sum(p * p, axis=-1, keepdims=True))
    pn = p / jnp.maximum(p_norm, 1e-12)

    d = xb.shape[1]
    ones_col = jnp.ones((d, 1), dtype=jnp.float32)
    ssum = jnp.dot(w2, ones_col, preferred_element_type=jnp.float32)
    w_norm = jnp.sqrt(ssum)
    inv_wn = 1.0 / jnp.maximum(w_norm, 2e-12)

    sims = jnp.dot(w, pn.T, preferred_element_type=jnp.float32) * inv_wn

    z = sims * tau
    m = jnp.max(z, axis=-1, keepdims=True)
    lse = m[:, 0] + jnp.log(jnp.sum(jnp.exp(z - m), axis=-1))
    k = p.shape[0]
    soft = (lse - math.log(k)) / tau

    novelty = jnp.exp(-gamma * soft)
    half_gate = 0.5 * (1.0 - alpha + alpha * novelty)
    o_ref[:] = w * half_gate[:, None]


def kernel(x, protos, log_tau, log_gamma, log_blend):
    B, T, D = x.shape
    K = protos.shape[0]
    rows = B * T
    x2 = x.reshape(rows, D)

    block_rows = 512
    grid = (rows // block_rows,)

    out = pl.pallas_call(
        _body,
        grid=grid,
        in_specs=[
            pl.BlockSpec(memory_space=pltpu.SMEM),
            pl.BlockSpec(memory_space=pltpu.SMEM),
            pl.BlockSpec(memory_space=pltpu.SMEM),
            pl.BlockSpec((block_rows, D), lambda i: (i, 0)),
            pl.BlockSpec((K, D), lambda i: (0, 0)),
        ],
        out_specs=pl.BlockSpec((block_rows, D), lambda i: (i, 0)),
        out_shape=jax.ShapeDtypeStruct((rows, D), x.dtype),
        compiler_params=pltpu.CompilerParams(
            dimension_semantics=("arbitrary",),
        ),
    )(
        log_tau.reshape(1),
        log_gamma.reshape(1),
        log_blend.reshape(1),
        x2,
        protos,
    )
    return out.reshape(B, T, D)
